# baseline (speedup 1.0000x reference)
import functools

import jax
import jax.numpy as jnp
from jax import lax
from jax.experimental import pallas as pl
from jax.experimental.pallas import tpu as pltpu
from jax.experimental.pallas import tpu_sc as plsc

N = 10000
E = 320000
D = 128
H = 64

NC = 2
NS = 16
NW = NC * NS
EPW = E // NW
CH = 128
NCHUNK = 80
EPAD = NW * NCHUNK * CH
NACC = 10048
RPT = N // NS
NP = 10240

_mesh = plsc.VectorSubcoreMesh(core_axis_name="c", subcore_axis_name="s")


@functools.partial(
    pl.kernel,
    out_type=jax.ShapeDtypeStruct((NC, 1, NP), jnp.float32),
    mesh=_mesh,
    scratch_types=[
        pltpu.VMEM((NCHUNK, CH), jnp.int32),
        pltpu.VMEM((128,), jnp.float32),
        pltpu.VMEM_SHARED((NP,), jnp.float32),
    ],
)
def _sc_degree(dst_hbm, zeros_hbm, out_hbm, dst_v, ones_v, acc):
    c = lax.axis_index("c")
    s = lax.axis_index("s")
    wid = c * NS + s
    for k in range(8):
        ones_v[pl.ds(k * 16, 16)] = jnp.ones((16,), jnp.float32)
    pltpu.sync_copy(dst_hbm.at[wid], dst_v)
    pltpu.sync_copy(zeros_hbm.at[pl.ds(s * 640, 640)], acc.at[pl.ds(s * 640, 640)])
    plsc.subcore_barrier()

    def body(j, carry):
        pltpu.sync_copy(ones_v, acc.at[dst_v.at[j]], add=True)
        return carry

    lax.fori_loop(0, NCHUNK, body, 0)
    plsc.subcore_barrier()
    pltpu.sync_copy(acc.at[pl.ds(s * 640, 640)], out_hbm.at[c, 0, pl.ds(s * 640, 640)])


@functools.partial(
    pl.kernel,
    out_type=jax.ShapeDtypeStruct((NC, N, H), jnp.float32),
    mesh=_mesh,
    scratch_types=[
        pltpu.VMEM((NCHUNK, CH), jnp.int32),
        pltpu.VMEM((NCHUNK, CH), jnp.int32),
        pltpu.VMEM((8, CH, H), jnp.float32),
        pltpu.VMEM_SHARED((NACC, H), jnp.float32),
        [pltpu.SemaphoreType.DMA] * 8,
        [pltpu.SemaphoreType.DMA] * 8,
    ],
    compiler_params=pltpu.CompilerParams(use_tc_tiling_on_sc=False),
)
def _sc_aggregate(hs_hbm, src_hbm, dst_hbm, zeros_hbm, out_hbm,
                  src_v, dst_v, rows_v, acc, gsem, ssem):
    c = lax.axis_index("c")
    s = lax.axis_index("s")
    wid = c * NS + s
    pltpu.sync_copy(src_hbm.at[wid], src_v)

    def gather(j, b):
        return pltpu.async_copy(hs_hbm.at[src_v.at[j]], rows_v.at[b], gsem[b])

    def gather_wait(j, b):
        pltpu.make_async_copy(hs_hbm.at[src_v.at[j]], rows_v.at[b], gsem[b]).wait()

    def scatter(j, b):
        return pltpu.async_copy(rows_v.at[b], acc.at[dst_v.at[j]], ssem[b],
                                add=True)

    def scatter_wait(j, b):
        pltpu.make_async_copy(rows_v.at[b], acc.at[dst_v.at[j]], ssem[b]).wait()

    NB = 8
    for b in range(NB):
        gather(b, b)

    pltpu.sync_copy(dst_hbm.at[wid], dst_v)
    pltpu.sync_copy(zeros_hbm.at[pl.ds(s * 624, 624)], acc.at[pl.ds(s * 624, 624)])

    @pl.when(s == NS - 1)
    def _():
        pltpu.sync_copy(zeros_hbm.at[pl.ds(9984, 16)], acc.at[pl.ds(9984, 16)])

    plsc.subcore_barrier()

    NT = NCHUNK // NB

    def body(t, carry):
        j0 = t * NB
        for b in range(NB):
            gather_wait(j0 + b, b)
            scatter(j0 + b, b)
        for b in range(NB):
            scatter_wait(j0 + b, b)
            gather(j0 + NB + b, b)
        return carry

    lax.fori_loop(0, NT - 1, body, 0)
    j0 = (NT - 1) * NB
    for b in range(NB):
        gather_wait(j0 + b, b)
        scatter(j0 + b, b)
    for b in range(NB):
        scatter_wait(j0 + b, b)

    plsc.subcore_barrier()
    pltpu.sync_copy(acc.at[pl.ds(s * 624, 624)], out_hbm.at[c, pl.ds(s * 624, 624)])

    @pl.when(s == NS - 1)
    def _():
        pltpu.sync_copy(acc.at[pl.ds(9984, 16)], out_hbm.at[c, pl.ds(9984, 16)])


NPAIR = N // 2


def _tc_phase_a(x_ref, w1b_ref, dis_ref, hs1_ref):
    h = jnp.dot(x_ref[...], w1b_ref[...], preferred_element_type=jnp.float32)
    hs1_ref[...] = h * dis_ref[...]


def _tc_phase_b(s1_ref, hs1_ref, dis_ref, b1_ref, w2b_ref, hs2_ref):
    dis = dis_ref[...]
    h1 = jnp.maximum(
        dis * (s1_ref[0] + s1_ref[1] + hs1_ref[...]) + b1_ref[...], 0.0)
    hs2_ref[...] = jnp.dot(h1, w2b_ref[...], preferred_element_type=jnp.float32) * dis


def _tc_phase_c(s2_ref, hs2_ref, dis_ref, b2_ref, wlb_ref, blb_ref, out_ref):
    dis = dis_ref[...]
    h2 = jnp.maximum(
        dis * (s2_ref[0] + s2_ref[1] + hs2_ref[...]) + b2_ref[...], 0.0)
    out_ref[...] = jnp.dot(h2, wlb_ref[...], preferred_element_type=jnp.float32) + blb_ref[...]


_phase_a = pl.pallas_call(
    _tc_phase_a,
    out_shape=jax.ShapeDtypeStruct((NPAIR, 2 * H), jnp.float32),
)
_phase_b = pl.pallas_call(
    _tc_phase_b,
    out_shape=jax.ShapeDtypeStruct((NPAIR, 2 * H), jnp.float32),
)
_phase_c = pl.pallas_call(
    _tc_phase_c,
    out_shape=jax.ShapeDtypeStruct((NPAIR, 2), jnp.float32),
)


def _blockdiag(w):
    k, m = w.shape
    z = jnp.zeros((k, m), jnp.float32)
    return jnp.concatenate(
        [jnp.concatenate([w, z], axis=1), jnp.concatenate([z, w], axis=1)],
        axis=0)


def kernel(x, edge_index, W1, b1, W2, b2, Wl, bl):
    npad = EPAD - E
    src = jnp.concatenate(
        [edge_index[0].astype(jnp.int32), jnp.zeros((npad,), jnp.int32)]
    ).reshape(NW, NCHUNK, CH)
    dst = jnp.concatenate(
        [edge_index[1].astype(jnp.int32), jnp.full((npad,), N, jnp.int32)]
    ).reshape(NW, NCHUNK, CH)
    zeros1 = jnp.zeros((NP,), jnp.float32)
    zeros2 = jnp.zeros((NACC, H), jnp.float32)

    w1b = _blockdiag(W1)
    w2b = _blockdiag(W2)
    wlb = _blockdiag(Wl)
    b1p = jnp.concatenate([b1, b1]).reshape(1, 2 * H)
    b2p = jnp.concatenate([b2, b2]).reshape(1, 2 * H)
    blp = jnp.concatenate([bl, bl]).reshape(1, 2)

    degp = _sc_degree(dst, zeros1)
    deg = degp[0, 0, :N] + degp[1, 0, :N] + 1.0
    dis = lax.rsqrt(deg)
    dis_pk = jnp.broadcast_to(dis.reshape(NPAIR, 2, 1),
                              (NPAIR, 2, H)).reshape(NPAIR, 2 * H)

    x_pk = x.reshape(NPAIR, 2 * D)
    hs1 = _phase_a(x_pk, w1b, dis_pk)

    s1 = _sc_aggregate(hs1.reshape(N, H), src, dst, zeros2)
    hs2 = _phase_b(s1.reshape(NC, NPAIR, 2 * H), hs1, dis_pk, b1p, w2b)

    s2 = _sc_aggregate(hs2.reshape(N, H), src, dst, zeros2)
    out = _phase_c(s2.reshape(NC, NPAIR, 2 * H), hs2, dis_pk, b2p, wlb, blp)
    return out.reshape(N, 1)


# speedup vs baseline: 1.0038x; 1.0038x over previous
import functools

import jax
import jax.numpy as jnp
from jax import lax
from jax.experimental import pallas as pl
from jax.experimental.pallas import tpu as pltpu
from jax.experimental.pallas import tpu_sc as plsc

N = 10000
E = 320000
D = 128
H = 64

NC = 2
NS = 16
NW = NC * NS
EPW = E // NW
CH = 128
NCHUNK = 80
EPAD = NW * NCHUNK * CH
NACC = 10048
RPT = N // NS
NP = 10240

_mesh = plsc.VectorSubcoreMesh(core_axis_name="c", subcore_axis_name="s")


@functools.partial(
    pl.kernel,
    out_type=jax.ShapeDtypeStruct((NC, 1, NP), jnp.float32),
    mesh=_mesh,
    scratch_types=[
        pltpu.VMEM((NCHUNK, CH), jnp.int32),
        pltpu.VMEM((128,), jnp.float32),
        pltpu.VMEM_SHARED((NP,), jnp.float32),
    ],
)
def _sc_degree(dst_hbm, zeros_hbm, out_hbm, dst_v, ones_v, acc):
    c = lax.axis_index("c")
    s = lax.axis_index("s")
    wid = c * NS + s
    for k in range(8):
        ones_v[pl.ds(k * 16, 16)] = jnp.ones((16,), jnp.float32)
    pltpu.sync_copy(dst_hbm.at[wid], dst_v)
    pltpu.sync_copy(zeros_hbm.at[pl.ds(s * 640, 640)], acc.at[pl.ds(s * 640, 640)])
    plsc.subcore_barrier()

    def body(j, carry):
        pltpu.sync_copy(ones_v, acc.at[dst_v.at[j]], add=True)
        return carry

    lax.fori_loop(0, NCHUNK, body, 0)
    plsc.subcore_barrier()
    pltpu.sync_copy(acc.at[pl.ds(s * 640, 640)], out_hbm.at[c, 0, pl.ds(s * 640, 640)])


@functools.partial(
    pl.kernel,
    out_type=jax.ShapeDtypeStruct((NC, N, H), jnp.float32),
    mesh=_mesh,
    scratch_types=[
        pltpu.VMEM((NCHUNK, CH), jnp.int32),
        pltpu.VMEM((NCHUNK, CH), jnp.int32),
        pltpu.VMEM((8, CH, H), jnp.float32),
        pltpu.VMEM_SHARED((NACC, H), jnp.float32),
        [pltpu.SemaphoreType.DMA] * 8,
        [pltpu.SemaphoreType.DMA] * 8,
    ],
    compiler_params=pltpu.CompilerParams(use_tc_tiling_on_sc=False),
)
def _sc_aggregate(hs_hbm, src_hbm, dst_hbm, zeros_hbm, out_hbm,
                  src_v, dst_v, rows_v, acc, gsem, ssem):
    c = lax.axis_index("c")
    s = lax.axis_index("s")
    wid = c * NS + s
    pltpu.sync_copy(src_hbm.at[wid], src_v)

    def gather(j, b):
        return pltpu.async_copy(hs_hbm.at[src_v.at[j]], rows_v.at[b], gsem[b])

    def gather_wait(j, b):
        pltpu.make_async_copy(hs_hbm.at[src_v.at[j]], rows_v.at[b], gsem[b]).wait()

    def scatter(j, b):
        return pltpu.async_copy(rows_v.at[b], acc.at[dst_v.at[j]], ssem[b],
                                add=True)

    def scatter_wait(j, b):
        pltpu.make_async_copy(rows_v.at[b], acc.at[dst_v.at[j]], ssem[b]).wait()

    NB = 8
    for b in range(NB):
        gather(b, b)

    pltpu.sync_copy(dst_hbm.at[wid], dst_v)
    pltpu.sync_copy(zeros_hbm.at[pl.ds(s * 624, 624)], acc.at[pl.ds(s * 624, 624)])

    @pl.when(s == NS - 1)
    def _():
        pltpu.sync_copy(zeros_hbm.at[pl.ds(9984, 16)], acc.at[pl.ds(9984, 16)])

    plsc.subcore_barrier()

    NT = NCHUNK // NB

    def body(t, carry):
        j0 = t * NB
        for b in range(NB):
            gather_wait(j0 + b, b)
            scatter(j0 + b, b)
        for b in range(NB):
            scatter_wait(j0 + b, b)
            gather(j0 + NB + b, b)
        return carry

    lax.fori_loop(0, NT - 1, body, 0)
    j0 = (NT - 1) * NB
    for b in range(NB):
        gather_wait(j0 + b, b)
        scatter(j0 + b, b)
    for b in range(NB):
        scatter_wait(j0 + b, b)

    plsc.subcore_barrier()
    pltpu.sync_copy(acc.at[pl.ds(s * 624, 624)], out_hbm.at[c, pl.ds(s * 624, 624)])

    @pl.when(s == NS - 1)
    def _():
        pltpu.sync_copy(acc.at[pl.ds(9984, 16)], out_hbm.at[c, pl.ds(9984, 16)])


NPAIR = N // 2


def _tc_phase_a(x_ref, w1b_ref, dis_ref, hs1_ref):
    h = jnp.dot(x_ref[...], w1b_ref[...], preferred_element_type=jnp.float32)
    hs1_ref[...] = h * dis_ref[...]


def _tc_phase_b(s1_ref, hs1_ref, dis_ref, b1_ref, w2b_ref, hs2_ref):
    dis = dis_ref[...]
    h1 = jnp.maximum(
        dis * (s1_ref[0] + s1_ref[1] + hs1_ref[...]) + b1_ref[...], 0.0)
    hs2_ref[...] = jnp.dot(h1, w2b_ref[...], preferred_element_type=jnp.float32) * dis


def _tc_phase_c(s2_ref, hs2_ref, dis_ref, b2_ref, wlb_ref, blb_ref, out_ref):
    dis = dis_ref[...]
    h2 = jnp.maximum(
        dis * (s2_ref[0] + s2_ref[1] + hs2_ref[...]) + b2_ref[...], 0.0)
    out_ref[...] = jnp.dot(h2, wlb_ref[...], preferred_element_type=jnp.float32) + blb_ref[...]


_phase_a = pl.pallas_call(
    _tc_phase_a,
    out_shape=jax.ShapeDtypeStruct((NPAIR, 2 * H), jnp.float32),
)
_phase_b = pl.pallas_call(
    _tc_phase_b,
    out_shape=jax.ShapeDtypeStruct((NPAIR, 2 * H), jnp.float32),
)
_phase_c = pl.pallas_call(
    _tc_phase_c,
    out_shape=jax.ShapeDtypeStruct((NPAIR, 2), jnp.float32),
)


def _blockdiag(w):
    k, m = w.shape
    z = jnp.zeros((k, m), jnp.float32)
    return jnp.concatenate(
        [jnp.concatenate([w, z], axis=1), jnp.concatenate([z, w], axis=1)],
        axis=0)


def kernel(x, edge_index, W1, b1, W2, b2, Wl, bl):
    npad = EPAD - E
    src = jnp.concatenate(
        [edge_index[0].astype(jnp.int32), jnp.zeros((npad,), jnp.int32)]
    ).reshape(NW, NCHUNK, CH)
    dst = jnp.concatenate(
        [edge_index[1].astype(jnp.int32), N + (jnp.arange(npad, dtype=jnp.int32) % (NACC - N))]
    ).reshape(NW, NCHUNK, CH)
    zeros1 = jnp.zeros((NP,), jnp.float32)
    zeros2 = jnp.zeros((NACC, H), jnp.float32)

    w1b = _blockdiag(W1)
    w2b = _blockdiag(W2)
    wlb = _blockdiag(Wl)
    b1p = jnp.concatenate([b1, b1]).reshape(1, 2 * H)
    b2p = jnp.concatenate([b2, b2]).reshape(1, 2 * H)
    blp = jnp.concatenate([bl, bl]).reshape(1, 2)

    degp = _sc_degree(dst, zeros1)
    deg = degp[0, 0, :N] + degp[1, 0, :N] + 1.0
    dis = lax.rsqrt(deg)
    dis_pk = jnp.broadcast_to(dis.reshape(NPAIR, 2, 1),
                              (NPAIR, 2, H)).reshape(NPAIR, 2 * H)

    x_pk = x.reshape(NPAIR, 2 * D)
    hs1 = _phase_a(x_pk, w1b, dis_pk)

    s1 = _sc_aggregate(hs1.reshape(N, H), src, dst, zeros2)
    hs2 = _phase_b(s1.reshape(NC, NPAIR, 2 * H), hs1, dis_pk, b1p, w2b)

    s2 = _sc_aggregate(hs2.reshape(N, H), src, dst, zeros2)
    out = _phase_c(s2.reshape(NC, NPAIR, 2 * H), hs2, dis_pk, b2p, wlb, blp)
    return out.reshape(N, 1)


# speedup vs baseline: 3.0680x; 3.0564x over previous
import functools

import jax
import jax.numpy as jnp
from jax import lax
from jax.experimental import pallas as pl
from jax.experimental.pallas import tpu as pltpu
from jax.experimental.pallas import tpu_sc as plsc

N = 10000
E = 320000
D = 128
H = 64

NC = 2
NS = 16
NW = NC * NS
EPW = E // NW
CH = 125
NCHUNK = EPW // CH
RPT = N // NS
NP = 10240

_mesh = plsc.VectorSubcoreMesh(core_axis_name="c", subcore_axis_name="s")


@functools.partial(
    pl.kernel,
    out_type=jax.ShapeDtypeStruct((NC, 1, NP), jnp.float32),
    mesh=_mesh,
    scratch_types=[
        pltpu.VMEM((NCHUNK, CH), jnp.int32),
        pltpu.VMEM((128,), jnp.float32),
        pltpu.VMEM_SHARED((NP,), jnp.float32),
    ],
)
def _sc_degree(dst_hbm, zeros_hbm, out_hbm, dst_v, ones_v, acc):
    c = lax.axis_index("c")
    s = lax.axis_index("s")
    wid = c * NS + s
    for k in range(8):
        ones_v[pl.ds(k * 16, 16)] = jnp.ones((16,), jnp.float32)
    pltpu.sync_copy(dst_hbm.at[wid], dst_v)
    pltpu.sync_copy(zeros_hbm.at[pl.ds(s * 640, 640)], acc.at[pl.ds(s * 640, 640)])
    plsc.subcore_barrier()

    def body(j, carry):
        pltpu.sync_copy(ones_v.at[pl.ds(0, CH)], acc.at[dst_v.at[j]], add=True)
        return carry

    lax.fori_loop(0, NCHUNK, body, 0)
    plsc.subcore_barrier()
    pltpu.sync_copy(acc.at[pl.ds(s * 640, 640)], out_hbm.at[c, 0, pl.ds(s * 640, 640)])


@functools.partial(
    pl.kernel,
    out_type=jax.ShapeDtypeStruct((NC, N, H), jnp.float32),
    mesh=_mesh,
    scratch_types=[
        pltpu.VMEM((NCHUNK, CH), jnp.int32),
        pltpu.VMEM((NCHUNK, CH), jnp.int32),
        pltpu.VMEM((8, CH, H), jnp.float32),
        pltpu.VMEM_SHARED((N, H), jnp.float32),
        [pltpu.SemaphoreType.DMA] * 8,
        [pltpu.SemaphoreType.DMA] * 8,
    ],
    compiler_params=pltpu.CompilerParams(use_tc_tiling_on_sc=False),
)
def _sc_aggregate(hs_hbm, src_hbm, dst_hbm, zeros_hbm, out_hbm,
                  src_v, dst_v, rows_v, acc, gsem, ssem):
    c = lax.axis_index("c")
    s = lax.axis_index("s")
    wid = c * NS + s
    pltpu.sync_copy(src_hbm.at[wid], src_v)

    def gather(j, b):
        return pltpu.async_copy(hs_hbm.at[src_v.at[j]], rows_v.at[b], gsem[b])

    def gather_wait(j, b):
        pltpu.make_async_copy(hs_hbm.at[src_v.at[j]], rows_v.at[b], gsem[b]).wait()

    def scatter(j, b):
        return pltpu.async_copy(rows_v.at[b], acc.at[dst_v.at[j]], ssem[b],
                                add=True)

    def scatter_wait(j, b):
        pltpu.make_async_copy(rows_v.at[b], acc.at[dst_v.at[j]], ssem[b]).wait()

    NB = 8
    for b in range(NB):
        gather(b, b)

    pltpu.sync_copy(dst_hbm.at[wid], dst_v)
    pltpu.sync_copy(zeros_hbm.at[pl.ds(s * 624, 624)], acc.at[pl.ds(s * 624, 624)])

    @pl.when(s == NS - 1)
    def _():
        pltpu.sync_copy(zeros_hbm.at[pl.ds(9984, 16)], acc.at[pl.ds(9984, 16)])

    plsc.subcore_barrier()

    NT = NCHUNK // NB

    def body(t, carry):
        j0 = t * NB
        for b in range(NB):
            gather_wait(j0 + b, b)
            scatter(j0 + b, b)
        for b in range(NB):
            scatter_wait(j0 + b, b)
            gather(j0 + NB + b, b)
        return carry

    lax.fori_loop(0, NT - 1, body, 0)
    j0 = (NT - 1) * NB
    for b in range(NB):
        gather_wait(j0 + b, b)
        scatter(j0 + b, b)
    for b in range(NB):
        scatter_wait(j0 + b, b)

    plsc.subcore_barrier()
    pltpu.sync_copy(acc.at[pl.ds(s * 624, 624)], out_hbm.at[c, pl.ds(s * 624, 624)])

    @pl.when(s == NS - 1)
    def _():
        pltpu.sync_copy(acc.at[pl.ds(9984, 16)], out_hbm.at[c, pl.ds(9984, 16)])


NPAIR = N // 2


def _tc_phase_a(x_ref, w1b_ref, dis_ref, hs1_ref):
    h = jnp.dot(x_ref[...], w1b_ref[...], preferred_element_type=jnp.float32)
    hs1_ref[...] = h * dis_ref[...]


def _tc_phase_b(s1_ref, hs1_ref, dis_ref, b1_ref, w2b_ref, hs2_ref):
    dis = dis_ref[...]
    h1 = jnp.maximum(
        dis * (s1_ref[0] + s1_ref[1] + hs1_ref[...]) + b1_ref[...], 0.0)
    hs2_ref[...] = jnp.dot(h1, w2b_ref[...], preferred_element_type=jnp.float32) * dis


def _tc_phase_c(s2_ref, hs2_ref, dis_ref, b2_ref, wlb_ref, blb_ref, out_ref):
    dis = dis_ref[...]
    h2 = jnp.maximum(
        dis * (s2_ref[0] + s2_ref[1] + hs2_ref[...]) + b2_ref[...], 0.0)
    out_ref[...] = jnp.dot(h2, wlb_ref[...], preferred_element_type=jnp.float32) + blb_ref[...]


_phase_a = pl.pallas_call(
    _tc_phase_a,
    out_shape=jax.ShapeDtypeStruct((NPAIR, 2 * H), jnp.float32),
)
_phase_b = pl.pallas_call(
    _tc_phase_b,
    out_shape=jax.ShapeDtypeStruct((NPAIR, 2 * H), jnp.float32),
)
_phase_c = pl.pallas_call(
    _tc_phase_c,
    out_shape=jax.ShapeDtypeStruct((NPAIR, 2), jnp.float32),
)


def _blockdiag(w):
    k, m = w.shape
    z = jnp.zeros((k, m), jnp.float32)
    return jnp.concatenate(
        [jnp.concatenate([w, z], axis=1), jnp.concatenate([z, w], axis=1)],
        axis=0)


def kernel(x, edge_index, W1, b1, W2, b2, Wl, bl):
    src = edge_index[0].astype(jnp.int32).reshape(NW, NCHUNK, CH)
    dst = edge_index[1].astype(jnp.int32).reshape(NW, NCHUNK, CH)
    zeros1 = jnp.zeros((NP,), jnp.float32)
    zeros2 = jnp.zeros((N, H), jnp.float32)

    w1b = _blockdiag(W1)
    w2b = _blockdiag(W2)
    wlb = _blockdiag(Wl)
    b1p = jnp.concatenate([b1, b1]).reshape(1, 2 * H)
    b2p = jnp.concatenate([b2, b2]).reshape(1, 2 * H)
    blp = jnp.concatenate([bl, bl]).reshape(1, 2)

    degp = _sc_degree(dst, zeros1)
    deg = degp[0, 0, :N] + degp[1, 0, :N] + 1.0
    dis = lax.rsqrt(deg)
    dis_pk = jnp.broadcast_to(dis.reshape(NPAIR, 2, 1),
                              (NPAIR, 2, H)).reshape(NPAIR, 2 * H)

    x_pk = x.reshape(NPAIR, 2 * D)
    hs1 = _phase_a(x_pk, w1b, dis_pk)

    s1 = _sc_aggregate(hs1.reshape(N, H), src, dst, zeros2)
    hs2 = _phase_b(s1.reshape(NC, NPAIR, 2 * H), hs1, dis_pk, b1p, w2b)

    s2 = _sc_aggregate(hs2.reshape(N, H), src, dst, zeros2)
    out = _phase_c(s2.reshape(NC, NPAIR, 2 * H), hs2, dis_pk, b2p, wlb, blp)
    return out.reshape(N, 1)


# speedup vs baseline: 3.0705x; 1.0008x over previous
import functools

import jax
import jax.numpy as jnp
from jax import lax
from jax.experimental import pallas as pl
from jax.experimental.pallas import tpu as pltpu
from jax.experimental.pallas import tpu_sc as plsc

N = 10000
E = 320000
D = 128
H = 64

NC = 2
NS = 16
NW = NC * NS
EPW = E // NW
CH = 125
NCHUNK = EPW // CH
RPT = N // NS
NP = 10240

_mesh = plsc.VectorSubcoreMesh(core_axis_name="c", subcore_axis_name="s")


@functools.partial(
    pl.kernel,
    out_type=jax.ShapeDtypeStruct((NC, 1, NP), jnp.float32),
    mesh=_mesh,
    scratch_types=[
        pltpu.VMEM((NCHUNK, CH), jnp.int32),
        pltpu.VMEM((128,), jnp.float32),
        pltpu.VMEM_SHARED((NP,), jnp.float32),
    ],
    compiler_params=pltpu.CompilerParams(use_tc_tiling_on_sc=False),
)
def _sc_degree(dst_hbm, zeros_hbm, out_hbm, dst_v, ones_v, acc):
    c = lax.axis_index("c")
    s = lax.axis_index("s")
    wid = c * NS + s
    for k in range(8):
        ones_v[pl.ds(k * 16, 16)] = jnp.ones((16,), jnp.float32)
    pltpu.sync_copy(dst_hbm.at[wid], dst_v)
    pltpu.sync_copy(zeros_hbm.at[pl.ds(s * 640, 640)], acc.at[pl.ds(s * 640, 640)])
    plsc.subcore_barrier()

    def body(j, carry):
        pltpu.sync_copy(ones_v.at[pl.ds(0, CH)], acc.at[dst_v.at[j]], add=True)
        return carry

    lax.fori_loop(0, NCHUNK, body, 0)
    plsc.subcore_barrier()
    pltpu.sync_copy(acc.at[pl.ds(s * 640, 640)], out_hbm.at[c, 0, pl.ds(s * 640, 640)])


@functools.partial(
    pl.kernel,
    out_type=jax.ShapeDtypeStruct((NC, N, H), jnp.float32),
    mesh=_mesh,
    scratch_types=[
        pltpu.VMEM((NCHUNK, CH), jnp.int32),
        pltpu.VMEM((NCHUNK, CH), jnp.int32),
        pltpu.VMEM((8, CH, H), jnp.float32),
        pltpu.VMEM_SHARED((N, H), jnp.float32),
        [pltpu.SemaphoreType.DMA] * 8,
        [pltpu.SemaphoreType.DMA] * 8,
    ],
    compiler_params=pltpu.CompilerParams(use_tc_tiling_on_sc=False),
)
def _sc_aggregate(hs_hbm, src_hbm, dst_hbm, zeros_hbm, out_hbm,
                  src_v, dst_v, rows_v, acc, gsem, ssem):
    c = lax.axis_index("c")
    s = lax.axis_index("s")
    wid = c * NS + s
    pltpu.sync_copy(src_hbm.at[wid], src_v)

    def gather(j, b):
        return pltpu.async_copy(hs_hbm.at[src_v.at[j]], rows_v.at[b], gsem[b])

    def gather_wait(j, b):
        pltpu.make_async_copy(hs_hbm.at[src_v.at[j]], rows_v.at[b], gsem[b]).wait()

    def scatter(j, b):
        return pltpu.async_copy(rows_v.at[b], acc.at[dst_v.at[j]], ssem[b],
                                add=True)

    def scatter_wait(j, b):
        pltpu.make_async_copy(rows_v.at[b], acc.at[dst_v.at[j]], ssem[b]).wait()

    NB = 8
    for b in range(NB):
        gather(b, b)

    pltpu.sync_copy(dst_hbm.at[wid], dst_v)
    pltpu.sync_copy(zeros_hbm.at[pl.ds(s * 624, 624)], acc.at[pl.ds(s * 624, 624)])

    @pl.when(s == NS - 1)
    def _():
        pltpu.sync_copy(zeros_hbm.at[pl.ds(9984, 16)], acc.at[pl.ds(9984, 16)])

    plsc.subcore_barrier()

    NT = NCHUNK // NB

    def body(t, carry):
        j0 = t * NB
        for b in range(NB):
            gather_wait(j0 + b, b)
            scatter(j0 + b, b)
        for b in range(NB):
            scatter_wait(j0 + b, b)
            gather(j0 + NB + b, b)
        return carry

    lax.fori_loop(0, NT - 1, body, 0)
    j0 = (NT - 1) * NB
    for b in range(NB):
        gather_wait(j0 + b, b)
        scatter(j0 + b, b)
    for b in range(NB):
        scatter_wait(j0 + b, b)

    plsc.subcore_barrier()
    pltpu.sync_copy(acc.at[pl.ds(s * 624, 624)], out_hbm.at[c, pl.ds(s * 624, 624)])

    @pl.when(s == NS - 1)
    def _():
        pltpu.sync_copy(acc.at[pl.ds(9984, 16)], out_hbm.at[c, pl.ds(9984, 16)])


NPAIR = N // 2


def _tc_phase_a(x_ref, w1b_ref, dis_ref, hs1_ref):
    h = jnp.dot(x_ref[...], w1b_ref[...], preferred_element_type=jnp.float32)
    hs1_ref[...] = h * dis_ref[...]


def _tc_phase_b(s1_ref, hs1_ref, dis_ref, b1_ref, w2b_ref, hs2_ref):
    dis = dis_ref[...]
    h1 = jnp.maximum(
        dis * (s1_ref[0] + s1_ref[1] + hs1_ref[...]) + b1_ref[...], 0.0)
    hs2_ref[...] = jnp.dot(h1, w2b_ref[...], preferred_element_type=jnp.float32) * dis


def _tc_phase_c(s2_ref, hs2_ref, dis_ref, b2_ref, wlb_ref, blb_ref, out_ref):
    dis = dis_ref[...]
    h2 = jnp.maximum(
        dis * (s2_ref[0] + s2_ref[1] + hs2_ref[...]) + b2_ref[...], 0.0)
    out_ref[...] = jnp.dot(h2, wlb_ref[...], preferred_element_type=jnp.float32) + blb_ref[...]


_phase_a = pl.pallas_call(
    _tc_phase_a,
    out_shape=jax.ShapeDtypeStruct((NPAIR, 2 * H), jnp.float32),
)
_phase_b = pl.pallas_call(
    _tc_phase_b,
    out_shape=jax.ShapeDtypeStruct((NPAIR, 2 * H), jnp.float32),
)
_phase_c = pl.pallas_call(
    _tc_phase_c,
    out_shape=jax.ShapeDtypeStruct((NPAIR, 2), jnp.float32),
)


def _blockdiag(w):
    k, m = w.shape
    z = jnp.zeros((k, m), jnp.float32)
    return jnp.concatenate(
        [jnp.concatenate([w, z], axis=1), jnp.concatenate([z, w], axis=1)],
        axis=0)


def kernel(x, edge_index, W1, b1, W2, b2, Wl, bl):
    src = edge_index[0].astype(jnp.int32).reshape(NW, NCHUNK, CH)
    dst = edge_index[1].astype(jnp.int32).reshape(NW, NCHUNK, CH)
    zeros1 = jnp.zeros((NP,), jnp.float32)
    zeros2 = jnp.zeros((N, H), jnp.float32)

    w1b = _blockdiag(W1)
    w2b = _blockdiag(W2)
    wlb = _blockdiag(Wl)
    b1p = jnp.concatenate([b1, b1]).reshape(1, 2 * H)
    b2p = jnp.concatenate([b2, b2]).reshape(1, 2 * H)
    blp = jnp.concatenate([bl, bl]).reshape(1, 2)

    degp = _sc_degree(dst, zeros1)
    deg = degp[0, 0, :N] + degp[1, 0, :N] + 1.0
    dis = lax.rsqrt(deg)
    dis_pk = jnp.broadcast_to(dis.reshape(NPAIR, 2, 1),
                              (NPAIR, 2, H)).reshape(NPAIR, 2 * H)

    x_pk = x.reshape(NPAIR, 2 * D)
    hs1 = _phase_a(x_pk, w1b, dis_pk)

    s1 = _sc_aggregate(hs1.reshape(N, H), src, dst, zeros2)
    hs2 = _phase_b(s1.reshape(NC, NPAIR, 2 * H), hs1, dis_pk, b1p, w2b)

    s2 = _sc_aggregate(hs2.reshape(N, H), src, dst, zeros2)
    out = _phase_c(s2.reshape(NC, NPAIR, 2 * H), hs2, dis_pk, b2p, wlb, blp)
    return out.reshape(N, 1)
